# Initial kernel scaffold; baseline (speedup 1.0000x reference)
import jax
import jax.numpy as jnp
from jax.experimental import pallas as pl


def kernel(x, weight):
    raise NotImplementedError("write your pallas kernel here")


# speedup vs baseline: 3.6065x; 3.6065x over previous
import functools

import jax
import jax.numpy as jnp
from jax import lax
from jax.experimental import pallas as pl
from jax.experimental.pallas import tpu as pltpu
from jax.experimental.pallas import tpu_sc as plsc

N_TABLES = 26
NUM_EMB = 100000
EMB_DIM = 32

_NC, _NS = 2, 16
_NW = _NC * _NS
_NBUF = 2


def _wide_embed(x_flat, weight, *, total):
    bc = total // _NW
    cc = bc // _NBUF

    mesh = plsc.VectorSubcoreMesh(core_axis_name="c", subcore_axis_name="s")

    @functools.partial(
        pl.kernel,
        mesh=mesh,
        out_type=jax.ShapeDtypeStruct((total, N_TABLES * EMB_DIM), jnp.float32),
        scratch_types=[
            pltpu.VMEM((bc,), jnp.int32),
            pltpu.VMEM((_NBUF, cc, EMB_DIM), jnp.float32),
            pltpu.SemaphoreType.DMA,
            pltpu.SemaphoreType.DMA,
            pltpu.SemaphoreType.DMA,
            pltpu.SemaphoreType.DMA,
        ],
        compiler_params=pltpu.CompilerParams(use_tc_tiling_on_sc=False),
    )
    def k(w_hbm, idx_hbm, out_hbm, idx_v, rows_v, gsem0, gsem1, wsem0, wsem1):
        wid = lax.axis_index("s") * _NC + lax.axis_index("c")
        base = wid * bc
        pltpu.sync_copy(idx_hbm.at[pl.ds(base, bc)], idx_v)

        gsems = (gsem0, gsem1)
        wsems = (wsem0, wsem1)

        def gather(t, b):
            pltpu.async_copy(
                w_hbm.at[t].at[idx_v.at[pl.ds(b * cc, cc)]],
                rows_v.at[b],
                gsems[b],
            )

        def gather_wait(t, b):
            pltpu.make_async_copy(
                w_hbm.at[t].at[idx_v.at[pl.ds(b * cc, cc)]],
                rows_v.at[b],
                gsems[b],
            ).wait()

        def wb(t, b):
            return pltpu.make_async_copy(
                rows_v.at[b],
                out_hbm.at[
                    pl.ds(base + b * cc, cc),
                    pl.ds(t * EMB_DIM, EMB_DIM),
                ],
                wsems[b],
            )

        @pl.loop(0, N_TABLES)
        def table_loop(t):
            @pl.when(t > 0)
            def _():
                for b in range(_NBUF):
                    wb(t - 1, b).wait()

            for b in range(_NBUF):
                gather(t, b)
            for b in range(_NBUF):
                gather_wait(t, b)
                wb(t, b).start()

        for b in range(_NBUF):
            wb(N_TABLES - 1, b).wait()

    return k(weight, x_flat)


def kernel(x, weight):
    B, T = x.shape
    total = B * T
    out = _wide_embed(x.reshape(total), weight, total=total)
    return out.reshape(B, T, N_TABLES * EMB_DIM)
